# Initial kernel scaffold; baseline (speedup 1.0000x reference)
import jax
import jax.numpy as jnp
from jax.experimental import pallas as pl


def kernel(feat_seq, para, pW1, pb1, pW2, pb2, fW, fb, emb_table, sW1, sb1, sW2, sb2):
    raise NotImplementedError("write your pallas kernel here")


# speedup vs baseline: 3.8080x; 3.8080x over previous
import jax
import jax.numpy as jnp
import numpy as np
from jax.experimental import pallas as pl

B = 16
T = 2048
DIN = 256
HID = 256
K = 16
TT = 256


def _pe_kernel(para_ref, pW1_ref, pb1_ref, pW2_ref, pb2_ref, fWb_ref, fb_ref,
               c_ref):
    h = jnp.dot(para_ref[...], pW1_ref[...],
                preferred_element_type=jnp.float32) + pb1_ref[...]
    h = jnp.maximum(h, 0.0)
    pe = jnp.dot(h, pW2_ref[...],
                 preferred_element_type=jnp.float32) + pb2_ref[...]
    c_ref[...] = jnp.dot(pe, fWb_ref[...],
                         preferred_element_type=jnp.float32) + fb_ref[...]


def _score_kernel(feat_ref, emb_ref, g_ref, c_ref, fWt_ref, Mmid_ref,
                  Mprod_ref, Memb_ref, sb1_ref, sW2_ref, y_ref):
    feat = feat_ref[...].reshape(B * TT, DIN)
    emb = emb_ref[...]
    mid = jnp.dot(feat, fWt_ref[...], preferred_element_type=jnp.float32)
    mid = mid.reshape(B, TT, HID) + c_ref[...][:, None, :]
    t1 = jnp.dot(mid.reshape(B * TT, HID), Mmid_ref[...],
                 preferred_element_type=jnp.float32)
    prod = mid * emb[None, :, :]
    t2 = jnp.dot(prod.reshape(B * TT, HID), Mprod_ref[...],
                 preferred_element_type=jnp.float32)
    ec = jnp.dot(emb, Memb_ref[...], preferred_element_type=jnp.float32)
    h = (t1 + t2).reshape(B, TT, HID) + ec[None, :, :] + sb1_ref[...][None, :, :]
    h = jnp.maximum(h, 0.0)
    s = jnp.dot(h.reshape(B * TT, HID), sW2_ref[...],
                preferred_element_type=jnp.float32)
    y_ref[...] = s.reshape(B, TT) + g_ref[...]


def _topk_kernel(y_ref, out_ref):
    y = y_ref[...]
    iota = jax.lax.broadcasted_iota(jnp.int32, (B, T), 1)
    neg = jnp.float32(-jnp.inf)
    sel = []
    for _ in range(K - 2):
        m = jnp.max(y, axis=1, keepdims=True)
        cand = jnp.where(y == m, iota, T)
        a = jnp.min(cand, axis=1)
        sel.append(a)
        y = jnp.where(iota == a[:, None], neg, y)
    vals = jnp.concatenate(
        [jnp.zeros((B, 1), jnp.int32)] +
        [a[:, None] for a in sel] +
        [jnp.full((B, 1), T - 1, jnp.int32)], axis=1)
    ranks = jnp.zeros((B, K), jnp.int32)
    for j in range(K):
        ranks = ranks + (vals[:, j][:, None] < vals).astype(jnp.int32)
    cols = [
        jnp.sum(jnp.where(ranks == j, vals, 0), axis=1, keepdims=True)
        for j in range(K)
    ]
    out_ref[...] = jnp.concatenate(cols, axis=1)


def kernel(feat_seq, para, pW1, pb1, pW2, pb2, fW, fb, emb_table, sW1, sb1,
           sW2, sb2):
    f32 = jnp.float32
    A = sW1[0:HID]
    Bm = sW1[HID:2 * HID]
    C = sW1[2 * HID:3 * HID]
    D = sW1[3 * HID:4 * HID]
    v = sW1[4 * HID]
    Mmid = A + C
    Mprod = D + jnp.ones((HID, 1), f32) * v[None, :]
    Memb = Bm - C
    fWt = fW[:DIN]
    fWb = fW[DIN:]

    g = jax.random.gumbel(jax.random.key(42), (B, T - 2), dtype=f32)
    g_pad = jnp.pad(g + sb2[0], ((0, 0), (1, 1)),
                    constant_values=-np.inf)
    emb_pad = jnp.pad(emb_table, ((1, 1), (0, 0)))

    c = pl.pallas_call(
        _pe_kernel,
        out_shape=jax.ShapeDtypeStruct((B, HID), f32),
    )(para, pW1, pb1.reshape(1, -1), pW2, pb2.reshape(1, -1), fWb,
      fb.reshape(1, -1))

    nt = T // TT
    y = pl.pallas_call(
        _score_kernel,
        grid=(nt,),
        in_specs=[
            pl.BlockSpec((B, TT, DIN), lambda i: (0, i, 0)),
            pl.BlockSpec((TT, HID), lambda i: (i, 0)),
            pl.BlockSpec((B, TT), lambda i: (0, i)),
            pl.BlockSpec((B, HID), lambda i: (0, 0)),
            pl.BlockSpec((DIN, HID), lambda i: (0, 0)),
            pl.BlockSpec((HID, HID), lambda i: (0, 0)),
            pl.BlockSpec((HID, HID), lambda i: (0, 0)),
            pl.BlockSpec((HID, HID), lambda i: (0, 0)),
            pl.BlockSpec((1, HID), lambda i: (0, 0)),
            pl.BlockSpec((HID, 1), lambda i: (0, 0)),
        ],
        out_specs=pl.BlockSpec((B, TT), lambda i: (0, i)),
        out_shape=jax.ShapeDtypeStruct((B, T), f32),
    )(feat_seq, emb_pad, g_pad, c, fWt, Mmid, Mprod, Memb,
      sb1.reshape(1, -1), sW2)

    idx = pl.pallas_call(
        _topk_kernel,
        out_shape=jax.ShapeDtypeStruct((B, K), jnp.int32),
    )(y)
    return idx


# speedup vs baseline: 4.0567x; 1.0653x over previous
import jax
import jax.numpy as jnp
import numpy as np
from jax.experimental import pallas as pl
from jax.experimental.pallas import tpu as pltpu

B = 16
T = 2048
DIN = 256
HID = 256
K = 16
TT = 256
NT = T // TT


def _fused_kernel(para_ref, pW1_ref, pb1_ref, pW2_ref, pb2_ref, fWb_ref,
                  fb_ref, sb2_ref, feat_ref, emb_ref, g_ref, fWt_ref,
                  Mmid_ref, Mprod_ref, Memb_ref, sb1_ref, sW2_ref,
                  out_ref, c_ref, y_ref):
    i = pl.program_id(0)

    @pl.when(i == 0)
    def _pe():
        h = jnp.dot(para_ref[...], pW1_ref[...],
                    preferred_element_type=jnp.float32) + pb1_ref[...]
        h = jnp.maximum(h, 0.0)
        pe = jnp.dot(h, pW2_ref[...],
                     preferred_element_type=jnp.float32) + pb2_ref[...]
        c_ref[...] = jnp.dot(pe, fWb_ref[...],
                             preferred_element_type=jnp.float32) + fb_ref[...]

    feat = feat_ref[...].reshape(B * TT, DIN)
    emb = emb_ref[...]
    mid = jnp.dot(feat, fWt_ref[...], preferred_element_type=jnp.float32)
    mid = mid.reshape(B, TT, HID) + c_ref[...][:, None, :]
    t1 = jnp.dot(mid.reshape(B * TT, HID), Mmid_ref[...],
                 preferred_element_type=jnp.float32)
    prod = mid * emb[None, :, :]
    t2 = jnp.dot(prod.reshape(B * TT, HID), Mprod_ref[...],
                 preferred_element_type=jnp.float32)
    ec = jnp.dot(emb, Memb_ref[...], preferred_element_type=jnp.float32)
    h = (t1 + t2).reshape(B, TT, HID) + ec[None, :, :] + sb1_ref[...][None, :, :]
    h = jnp.maximum(h, 0.0)
    s = jnp.dot(h.reshape(B * TT, HID), sW2_ref[...],
                preferred_element_type=jnp.float32)
    y_ref[:, pl.ds(i * TT, TT)] = s.reshape(B, TT) + g_ref[...] + sb2_ref[...]

    @pl.when(i == NT - 1)
    def _topk():
        y = y_ref[...]
        iota = jax.lax.broadcasted_iota(jnp.int32, (B, T), 1)
        neg = jnp.float32(-jnp.inf)
        sel = []
        for _ in range(K - 2):
            m = jnp.max(y, axis=1, keepdims=True)
            cand = jnp.where(y == m, iota, T)
            a = jnp.min(cand, axis=1)
            sel.append(a)
            y = jnp.where(iota == a[:, None], neg, y)
        vals = jnp.concatenate(
            [jnp.zeros((B, 1), jnp.int32)] +
            [a[:, None] for a in sel] +
            [jnp.full((B, 1), T - 1, jnp.int32)], axis=1)
        ranks = jnp.zeros((B, K), jnp.int32)
        for j in range(K):
            ranks = ranks + (vals[:, j][:, None] < vals).astype(jnp.int32)
        cols = [
            jnp.sum(jnp.where(ranks == j, vals, 0), axis=1, keepdims=True)
            for j in range(K)
        ]
        out_ref[...] = jnp.concatenate(cols, axis=1)


def kernel(feat_seq, para, pW1, pb1, pW2, pb2, fW, fb, emb_table, sW1, sb1,
           sW2, sb2):
    f32 = jnp.float32
    A = sW1[0:HID]
    Bm = sW1[HID:2 * HID]
    C = sW1[2 * HID:3 * HID]
    D = sW1[3 * HID:4 * HID]
    v = sW1[4 * HID]
    Mmid = A + C
    Mprod = D + jnp.ones((HID, 1), f32) * v[None, :]
    Memb = Bm - C
    fWt = fW[:DIN]
    fWb = fW[DIN:]

    g = jax.random.gumbel(jax.random.key(42), (B, T - 2), dtype=f32)
    g_pad = jnp.pad(g, ((0, 0), (1, 1)), constant_values=-np.inf)
    emb_pad = jnp.pad(emb_table, ((1, 1), (0, 0)))

    idx = pl.pallas_call(
        _fused_kernel,
        grid=(NT,),
        in_specs=[
            pl.BlockSpec((B, 2), lambda i: (0, 0)),
            pl.BlockSpec((2, 2 * HID), lambda i: (0, 0)),
            pl.BlockSpec((1, 2 * HID), lambda i: (0, 0)),
            pl.BlockSpec((2 * HID, HID), lambda i: (0, 0)),
            pl.BlockSpec((1, HID), lambda i: (0, 0)),
            pl.BlockSpec((HID, HID), lambda i: (0, 0)),
            pl.BlockSpec((1, HID), lambda i: (0, 0)),
            pl.BlockSpec((1, 1), lambda i: (0, 0)),
            pl.BlockSpec((B, TT, DIN), lambda i: (0, i, 0)),
            pl.BlockSpec((TT, HID), lambda i: (i, 0)),
            pl.BlockSpec((B, TT), lambda i: (0, i)),
            pl.BlockSpec((DIN, HID), lambda i: (0, 0)),
            pl.BlockSpec((HID, HID), lambda i: (0, 0)),
            pl.BlockSpec((HID, HID), lambda i: (0, 0)),
            pl.BlockSpec((HID, HID), lambda i: (0, 0)),
            pl.BlockSpec((1, HID), lambda i: (0, 0)),
            pl.BlockSpec((HID, 1), lambda i: (0, 0)),
        ],
        out_specs=pl.BlockSpec((B, K), lambda i: (0, 0)),
        out_shape=jax.ShapeDtypeStruct((B, K), jnp.int32),
        scratch_shapes=[
            pltpu.VMEM((B, HID), f32),
            pltpu.VMEM((B, T), f32),
        ],
    )(para, pW1, pb1.reshape(1, -1), pW2, pb2.reshape(1, -1), fWb,
      fb.reshape(1, -1), sb2.reshape(1, 1), feat_seq, emb_pad, g_pad, fWt,
      Mmid, Mprod, Memb, sb1.reshape(1, -1), sW2)
    return idx
